# baseline (speedup 1.0000x reference)
import functools

import jax
import jax.numpy as jnp
from jax import lax
from jax.experimental import pallas as pl
from jax.experimental.pallas import tpu as pltpu
from jax.experimental.pallas import tpu_sc as plsc

BATCH = 4096
SEQ = 200
EMBED_DIM = 64
NUM_CORES = 2
NUM_SUBCORES = 16
NW = NUM_CORES * NUM_SUBCORES
BT = BATCH // NW
CT = EMBED_DIM // 8

_mesh = plsc.VectorSubcoreMesh(core_axis_name="c", subcore_axis_name="s")


@functools.partial(
    pl.kernel,
    out_type=jax.ShapeDtypeStruct((SEQ, CT, NW, 8, BT), jnp.float32),
    mesh=_mesh,
    compiler_params=pltpu.CompilerParams(
        use_tc_tiling_on_sc=False, needs_layout_passes=False),
    scratch_types=[
        pltpu.VMEM((SEQ, BT), jnp.int32),
        pltpu.VMEM((BT, EMBED_DIM), jnp.float32),
        pltpu.VMEM((BT, EMBED_DIM), jnp.float32),
        pltpu.VMEM((BT, EMBED_DIM), jnp.float32),
        pltpu.VMEM((BT, EMBED_DIM), jnp.float32),
        pltpu.VMEM((CT, 8, BT), jnp.float32),
        pltpu.VMEM((CT, 8, BT), jnp.float32),
        pltpu.VMEM((CT, 8, BT), jnp.float32),
        pltpu.VMEM((CT, 8, BT), jnp.float32),
        pltpu.SemaphoreType.DMA,
        pltpu.SemaphoreType.DMA,
        pltpu.SemaphoreType.DMA,
        pltpu.SemaphoreType.DMA,
        pltpu.SemaphoreType.DMA,
        pltpu.SemaphoreType.DMA,
        pltpu.SemaphoreType.DMA,
        pltpu.SemaphoreType.DMA,
    ],
)
def _sc_fused(idx_hbm, table_hbm, out_hbm, idx_v, rows0, rows1, rows2, rows3,
              t0, t1, t2, t3, gs0, gs1, gs2, gs3, ws0, ws1, ws2, ws3):
    wid = lax.axis_index("s") * NUM_CORES + lax.axis_index("c")
    rows = (rows0, rows1, rows2, rows3)
    tbuf = (t0, t1, t2, t3)
    gsem = (gs0, gs1, gs2, gs3)
    wsem = (ws0, ws1, ws2, ws3)

    pltpu.sync_copy(idx_hbm.at[wid], idx_v)

    def start_gather(s, b):
        pltpu.async_copy(table_hbm.at[idx_v.at[s]], rows[b], gsem[b])

    def wait_gather(s, b):
        pltpu.make_async_copy(table_hbm.at[idx_v.at[s]], rows[b], gsem[b]).wait()

    def start_write(s, b):
        pltpu.async_copy(tbuf[b], out_hbm.at[s, :, wid], wsem[b])

    def wait_write(s, b):
        pltpu.make_async_copy(tbuf[b], out_hbm.at[s, :, wid], wsem[b]).wait()

    iota = lax.iota(jnp.int32, 16)
    rowids = [iota + bg * 16 for bg in range(BT // 16)]

    def transpose(b):
        for ct in range(CT):
            for ci in range(8):
                col = jnp.full((16,), ct * 8 + ci, jnp.int32)
                for bg in range(BT // 16):
                    v = plsc.load_gather(rows[b], [rowids[bg], col])
                    tbuf[b][ct, ci, pl.ds(bg * 16, 16)] = v

    for b in range(4):
        start_gather(b, b)

    def body(i, _):
        s = 4 * i
        for b in range(4):
            sb = s + b

            @pl.when(sb >= 4)
            def _():
                wait_write(sb - 4, b)

            wait_gather(sb, b)
            transpose(b)

            @pl.when(sb + 4 < SEQ)
            def _():
                start_gather(sb + 4, b)

            start_write(sb, b)
        return _

    lax.fori_loop(0, SEQ // 4, body, None)
    for b in range(4):
        wait_write(SEQ - 4 + b, b)


def kernel(input_ids, word_embeddings):
    idx = input_ids.astype(jnp.int32).T.reshape(SEQ, NW, BT).transpose(1, 0, 2)
    out5 = _sc_fused(idx, word_embeddings)
    return out5.transpose(2, 4, 0, 1, 3).reshape(BATCH, SEQ, EMBED_DIM)


# speedup vs baseline: 1.3776x; 1.3776x over previous
import functools

import jax
import jax.numpy as jnp
from jax import lax
from jax.experimental import pallas as pl
from jax.experimental.pallas import tpu as pltpu
from jax.experimental.pallas import tpu_sc as plsc

BATCH = 4096
SEQ = 200
EMBED_DIM = 64
NUM_CORES = 2
NUM_SUBCORES = 16
NW = NUM_CORES * NUM_SUBCORES
BT = BATCH // NW
CT = EMBED_DIM // 8

_mesh = plsc.VectorSubcoreMesh(core_axis_name="c", subcore_axis_name="s")


@functools.partial(
    pl.kernel,
    out_type=jax.ShapeDtypeStruct((SEQ, CT, NW, 8, BT), jnp.float32),
    mesh=_mesh,
    compiler_params=pltpu.CompilerParams(
        use_tc_tiling_on_sc=False, needs_layout_passes=False),
    scratch_types=[
        pltpu.VMEM((SEQ, BT), jnp.int32),
        pltpu.VMEM((BT, EMBED_DIM), jnp.float32),
        pltpu.VMEM((BT, EMBED_DIM), jnp.float32),
        pltpu.VMEM((BT, EMBED_DIM), jnp.float32),
        pltpu.VMEM((BT, EMBED_DIM), jnp.float32),
        pltpu.VMEM((CT, 8, BT), jnp.float32),
        pltpu.VMEM((CT, 8, BT), jnp.float32),
        pltpu.VMEM((CT, 8, BT), jnp.float32),
        pltpu.VMEM((CT, 8, BT), jnp.float32),
        pltpu.SemaphoreType.DMA,
        pltpu.SemaphoreType.DMA,
        pltpu.SemaphoreType.DMA,
        pltpu.SemaphoreType.DMA,
        pltpu.SemaphoreType.DMA,
        pltpu.SemaphoreType.DMA,
        pltpu.SemaphoreType.DMA,
        pltpu.SemaphoreType.DMA,
    ],
)
def _sc_fused(idx_hbm, table_hbm, out_hbm, idx_v, rows0, rows1, rows2, rows3,
              t0, t1, t2, t3, gs0, gs1, gs2, gs3, ws0, ws1, ws2, ws3):
    wid = lax.axis_index("s") * NUM_CORES + lax.axis_index("c")
    rows = (rows0, rows1, rows2, rows3)
    tbuf = (t0, t1, t2, t3)
    gsem = (gs0, gs1, gs2, gs3)
    wsem = (ws0, ws1, ws2, ws3)

    pltpu.sync_copy(idx_hbm.at[wid], idx_v)

    def start_gather(s, b):
        pltpu.async_copy(table_hbm.at[idx_v.at[s]], rows[b], gsem[b])

    def wait_gather(s, b):
        pltpu.make_async_copy(table_hbm.at[idx_v.at[s]], rows[b], gsem[b]).wait()

    def start_write(s, b):
        pltpu.async_copy(tbuf[b], out_hbm.at[s, :, wid], wsem[b])

    def wait_write(s, b):
        pltpu.make_async_copy(tbuf[b], out_hbm.at[s, :, wid], wsem[b]).wait()

    iota = lax.iota(jnp.int32, 16)

    def transpose(b):
        @plsc.parallel_loop(0, (CT * 8 * BT) // 16, 1, unroll=8)
        def _t(j):
            bg = j & 7
            ci = (j >> 3) & 7
            ct = j >> 6
            rowv = iota + (bg << 4)
            colv = jnp.full((16,), ct * 8 + ci, jnp.int32)
            v = plsc.load_gather(rows[b], [rowv, colv])
            tbuf[b][ct, ci, pl.ds(bg * 16, 16)] = v

    for b in range(4):
        start_gather(b, b)

    def body(i, _):
        s = 4 * i
        for b in range(4):
            sb = s + b

            @pl.when(sb >= 4)
            def _():
                wait_write(sb - 4, b)

            wait_gather(sb, b)
            transpose(b)

            @pl.when(sb + 4 < SEQ)
            def _():
                start_gather(sb + 4, b)

            start_write(sb, b)
        return _

    lax.fori_loop(0, SEQ // 4, body, None)
    for b in range(4):
        wait_write(SEQ - 4 + b, b)


def kernel(input_ids, word_embeddings):
    idx = input_ids.astype(jnp.int32).T.reshape(SEQ, NW, BT).transpose(1, 0, 2)
    out5 = _sc_fused(idx, word_embeddings)
    return out5.transpose(2, 4, 0, 1, 3).reshape(BATCH, SEQ, EMBED_DIM)


# speedup vs baseline: 1.6652x; 1.2088x over previous
import functools

import jax
import jax.numpy as jnp
from jax import lax
from jax.experimental import pallas as pl
from jax.experimental.pallas import tpu as pltpu
from jax.experimental.pallas import tpu_sc as plsc

BATCH = 4096
SEQ = 200
EMBED_DIM = 64
NUM_CORES = 2
NUM_SUBCORES = 16
NW = NUM_CORES * NUM_SUBCORES
BT = BATCH // NW
CT = EMBED_DIM // 8

_mesh = plsc.VectorSubcoreMesh(core_axis_name="c", subcore_axis_name="s")


@functools.partial(
    pl.kernel,
    out_type=jax.ShapeDtypeStruct((SEQ, CT, NW, 8, BT), jnp.float32),
    mesh=_mesh,
    compiler_params=pltpu.CompilerParams(
        use_tc_tiling_on_sc=False, needs_layout_passes=False),
    scratch_types=[
        pltpu.VMEM((SEQ, BT), jnp.int32),
        pltpu.VMEM((BT, EMBED_DIM), jnp.float32),
        pltpu.VMEM((BT, EMBED_DIM), jnp.float32),
        pltpu.VMEM((BT, EMBED_DIM), jnp.float32),
        pltpu.VMEM((BT, EMBED_DIM), jnp.float32),
        pltpu.VMEM((CT, 8, BT), jnp.float32),
        pltpu.VMEM((CT, 8, BT), jnp.float32),
        pltpu.VMEM((CT, 8, BT), jnp.float32),
        pltpu.VMEM((CT, 8, BT), jnp.float32),
        pltpu.SemaphoreType.DMA,
        pltpu.SemaphoreType.DMA,
        pltpu.SemaphoreType.DMA,
        pltpu.SemaphoreType.DMA,
        pltpu.SemaphoreType.DMA,
        pltpu.SemaphoreType.DMA,
        pltpu.SemaphoreType.DMA,
        pltpu.SemaphoreType.DMA,
    ],
)
def _sc_fused(idx_hbm, table_hbm, out_hbm, idx_v, rows0, rows1, rows2, rows3,
              t0, t1, t2, t3, gs0, gs1, gs2, gs3, ws0, ws1, ws2, ws3):
    wid = lax.axis_index("s") * NUM_CORES + lax.axis_index("c")
    rows = (rows0, rows1, rows2, rows3)
    tbuf = (t0, t1, t2, t3)
    gsem = (gs0, gs1, gs2, gs3)
    wsem = (ws0, ws1, ws2, ws3)

    pltpu.sync_copy(idx_hbm.at[wid], idx_v)

    def start_gather(s, b):
        pltpu.async_copy(table_hbm.at[idx_v.at[s]], rows[b], gsem[b])

    def wait_gather(s, b):
        pltpu.make_async_copy(table_hbm.at[idx_v.at[s]], rows[b], gsem[b]).wait()

    def start_write(s, b):
        pltpu.async_copy(tbuf[b], out_hbm.at[s, :, wid], wsem[b])

    def wait_write(s, b):
        pltpu.make_async_copy(tbuf[b], out_hbm.at[s, :, wid], wsem[b]).wait()

    iota = lax.iota(jnp.int32, 16)
    rowvs = [iota + bg * 16 for bg in range(BT // 16)]

    def transpose(b):
        @plsc.parallel_loop(0, CT * 8, 1, unroll=2)
        def _t(j):
            ci = j & 7
            ct = j >> 3
            colv = jnp.full((16,), j, jnp.int32)
            for bg in range(BT // 16):
                v = plsc.load_gather(rows[b], [rowvs[bg], colv])
                tbuf[b][ct, ci, pl.ds(bg * 16, 16)] = v

    for b in range(4):
        start_gather(b, b)

    def body(i, _):
        s = 4 * i
        for b in range(4):
            sb = s + b

            @pl.when(sb >= 4)
            def _():
                wait_write(sb - 4, b)

            wait_gather(sb, b)
            transpose(b)

            @pl.when(sb + 4 < SEQ)
            def _():
                start_gather(sb + 4, b)

            start_write(sb, b)
        return _

    lax.fori_loop(0, SEQ // 4, body, None)
    for b in range(4):
        wait_write(SEQ - 4 + b, b)


def kernel(input_ids, word_embeddings):
    idx = input_ids.astype(jnp.int32).T.reshape(SEQ, NW, BT).transpose(1, 0, 2)
    out5 = _sc_fused(idx, word_embeddings)
    return out5.transpose(2, 4, 0, 1, 3).reshape(BATCH, SEQ, EMBED_DIM)


# speedup vs baseline: 1.6665x; 1.0007x over previous
import functools

import jax
import jax.numpy as jnp
from jax import lax
from jax.experimental import pallas as pl
from jax.experimental.pallas import tpu as pltpu
from jax.experimental.pallas import tpu_sc as plsc

BATCH = 4096
SEQ = 200
EMBED_DIM = 64
NUM_CORES = 2
NUM_SUBCORES = 16
NW = NUM_CORES * NUM_SUBCORES
BT = BATCH // NW
CT = EMBED_DIM // 8

_mesh = plsc.VectorSubcoreMesh(core_axis_name="c", subcore_axis_name="s")


@functools.partial(
    pl.kernel,
    out_type=jax.ShapeDtypeStruct((SEQ, CT, NW, 8, BT), jnp.float32),
    mesh=_mesh,
    compiler_params=pltpu.CompilerParams(
        use_tc_tiling_on_sc=False, needs_layout_passes=False),
    scratch_types=[
        pltpu.VMEM((SEQ // 8, 8, BT), jnp.int32),
        pltpu.VMEM((BT, EMBED_DIM), jnp.float32),
        pltpu.VMEM((BT, EMBED_DIM), jnp.float32),
        pltpu.VMEM((BT, EMBED_DIM), jnp.float32),
        pltpu.VMEM((BT, EMBED_DIM), jnp.float32),
        pltpu.VMEM((CT, 8, BT), jnp.float32),
        pltpu.VMEM((CT, 8, BT), jnp.float32),
        pltpu.VMEM((CT, 8, BT), jnp.float32),
        pltpu.VMEM((CT, 8, BT), jnp.float32),
        pltpu.SemaphoreType.DMA,
        pltpu.SemaphoreType.DMA,
        pltpu.SemaphoreType.DMA,
        pltpu.SemaphoreType.DMA,
        pltpu.SemaphoreType.DMA,
        pltpu.SemaphoreType.DMA,
        pltpu.SemaphoreType.DMA,
        pltpu.SemaphoreType.DMA,
    ],
)
def _sc_fused(idx_hbm, table_hbm, out_hbm, idx_v, rows0, rows1, rows2, rows3,
              t0, t1, t2, t3, gs0, gs1, gs2, gs3, ws0, ws1, ws2, ws3):
    wid = lax.axis_index("s") * NUM_CORES + lax.axis_index("c")
    rows = (rows0, rows1, rows2, rows3)
    tbuf = (t0, t1, t2, t3)
    gsem = (gs0, gs1, gs2, gs3)
    wsem = (ws0, ws1, ws2, ws3)

    pltpu.sync_copy(idx_hbm.at[:, wid], idx_v)

    def start_gather(s, b):
        pltpu.async_copy(table_hbm.at[idx_v.at[s >> 3, s & 7]], rows[b], gsem[b])

    def wait_gather(s, b):
        pltpu.make_async_copy(
            table_hbm.at[idx_v.at[s >> 3, s & 7]], rows[b], gsem[b]).wait()

    def start_write(s, b):
        pltpu.async_copy(tbuf[b], out_hbm.at[s, :, wid], wsem[b])

    def wait_write(s, b):
        pltpu.make_async_copy(tbuf[b], out_hbm.at[s, :, wid], wsem[b]).wait()

    iota = lax.iota(jnp.int32, 16)
    rowvs = [iota + bg * 16 for bg in range(BT // 16)]

    def transpose(b):
        @plsc.parallel_loop(0, CT * 8, 1, unroll=2)
        def _t(j):
            ci = j & 7
            ct = j >> 3
            colv = jnp.full((16,), j, jnp.int32)
            for bg in range(BT // 16):
                v = plsc.load_gather(rows[b], [rowvs[bg], colv])
                tbuf[b][ct, ci, pl.ds(bg * 16, 16)] = v

    for b in range(4):
        start_gather(b, b)

    def body(i, _):
        s = 4 * i
        for b in range(4):
            sb = s + b

            @pl.when(sb >= 4)
            def _():
                wait_write(sb - 4, b)

            wait_gather(sb, b)
            transpose(b)

            @pl.when(sb + 4 < SEQ)
            def _():
                start_gather(sb + 4, b)

            start_write(sb, b)
        return _

    lax.fori_loop(0, SEQ // 4, body, None)
    for b in range(4):
        wait_write(SEQ - 4 + b, b)


def kernel(input_ids, word_embeddings):
    idx = (input_ids.astype(jnp.int32).T
           .reshape(SEQ // 8, 8, NW, BT).transpose(0, 2, 1, 3))
    out5 = _sc_fused(idx, word_embeddings)
    return out5.transpose(2, 4, 0, 1, 3).reshape(BATCH, SEQ, EMBED_DIM)
